# Initial kernel scaffold; baseline (speedup 1.0000x reference)
import jax
import jax.numpy as jnp
from jax.experimental import pallas as pl


def kernel(x, edge_index, W_lin, b_lin, W_s2t, b_s2t, W_t2s, b_t2s):
    raise NotImplementedError("write your pallas kernel here")


# speedup vs baseline: 13.7668x; 13.7668x over previous
import functools

import jax
import jax.numpy as jnp
from jax import lax
from jax.experimental import pallas as pl
from jax.experimental.pallas import tpu as pltpu
from jax.experimental.pallas import tpu_sc as plsc

N_TILES = 16
LANES = 16


def _sc_aggregate(x, eidx, zrows, *, n_pad, chunk, n_chunks):
    rows_per_tile = n_pad // N_TILES
    d = x.shape[1]
    mesh = plsc.VectorSubcoreMesh(core_axis_name="c", subcore_axis_name="s")

    @functools.partial(
        pl.kernel,
        out_type=(
            jax.ShapeDtypeStruct((2, n_pad, d), jnp.bfloat16),
            jax.ShapeDtypeStruct((2 * N_TILES * n_pad,), jnp.float32),
        ),
        mesh=mesh,
        scratch_types=[
            pltpu.VMEM((n_chunks, chunk), jnp.int32),
            pltpu.VMEM((n_chunks, chunk), jnp.int32),
            pltpu.VMEM((chunk, d), jnp.bfloat16),
            pltpu.VMEM((chunk, d), jnp.bfloat16),
            pltpu.VMEM((n_pad,), jnp.float32),
            pltpu.VMEM_SHARED((n_pad, d), jnp.bfloat16),
            pltpu.SemaphoreType.DMA,
            pltpu.SemaphoreType.DMA,
        ],
        compiler_params=pltpu.CompilerParams(needs_layout_passes=False, use_tc_tiling_on_sc=False),
    )
    def body(x_hbm, eidx_hbm, zrows_hbm, agg_hbm, cnt_hbm,
             gidx_v, sidx_v, rows0, rows1, cnt_v, acc, sem0, sem1):
        c = lax.axis_index("c")
        s = lax.axis_index("s")
        r0 = s * rows_per_tile

        pltpu.sync_copy(zrows_hbm, acc.at[pl.ds(r0, rows_per_tile)])
        pltpu.sync_copy(eidx_hbm.at[c, s], gidx_v)
        pltpu.sync_copy(eidx_hbm.at[1 - c, s], sidx_v)

        zvec = jnp.zeros((LANES,), jnp.float32)

        @pl.loop(0, n_pad // LANES)
        def _(i):
            cnt_v[pl.ds(i * LANES, LANES)] = zvec

        plsc.subcore_barrier()

        ones = jnp.ones((LANES,), jnp.float32)

        pltpu.async_copy(x_hbm.at[gidx_v.at[0]], rows0, sem0)

        @pl.loop(0, n_chunks, step=2)
        def _(j):
            cp1 = pltpu.async_copy(x_hbm.at[gidx_v.at[j + 1]], rows1, sem1)

            @pl.loop(0, chunk // LANES)
            def _(k):
                idx = sidx_v[j, pl.ds(k * LANES, LANES)]
                plsc.addupdate_scatter(cnt_v, [idx], ones)

            pltpu.make_async_copy(x_hbm.at[pl.ds(0, chunk)], rows0, sem0).wait()
            pltpu.sync_copy(rows0, acc.at[sidx_v.at[j]], add=True)

            @pl.when(j + 2 < n_chunks)
            def _():
                pltpu.async_copy(x_hbm.at[gidx_v.at[j + 2]], rows0, sem0)

            @pl.loop(0, chunk // LANES)
            def _(k):
                idx = sidx_v[j + 1, pl.ds(k * LANES, LANES)]
                plsc.addupdate_scatter(cnt_v, [idx], ones)

            cp1.wait()
            pltpu.sync_copy(rows1, acc.at[sidx_v.at[j + 1]], add=True)

        pltpu.sync_copy(cnt_v, cnt_hbm.at[pl.ds((c * N_TILES + s) * n_pad, n_pad)])
        plsc.subcore_barrier()
        pltpu.sync_copy(acc.at[pl.ds(r0, rows_per_tile)],
                        agg_hbm.at[c, pl.ds(r0, rows_per_tile)])

    x = pltpu.with_memory_space_constraint(x, pltpu.HBM)
    eidx = pltpu.with_memory_space_constraint(eidx, pltpu.HBM)
    zrows = pltpu.with_memory_space_constraint(zrows, pltpu.HBM)
    return body(x, eidx, zrows)


def _tc_combine_body(x_ref, a0_ref, a1_ref, c0_ref, c1_ref,
                     wl_ref, ws_ref, wt_ref, bl_ref, bs_ref, bt_ref, o_ref):
    dn = (((1,), (1,)), ((), ()))
    x = x_ref[...]
    base = lax.dot_general(x, wl_ref[...], dn, preferred_element_type=jnp.float32)
    cnt0 = jnp.sum(c0_ref[...], axis=1, keepdims=True)
    r0 = 0.5 / jnp.maximum(cnt0, 1.0)
    m0 = lax.dot_general(a0_ref[...].astype(jnp.float32) * r0, ws_ref[...], dn,
                         preferred_element_type=jnp.float32)
    cnt1 = jnp.sum(c1_ref[...], axis=1, keepdims=True)
    r1 = 0.5 / jnp.maximum(cnt1, 1.0)
    m1 = lax.dot_general(a1_ref[...].astype(jnp.float32) * r1, wt_ref[...], dn,
                         preferred_element_type=jnp.float32)
    bias = bl_ref[...] + 0.5 * (bs_ref[...] + bt_ref[...])
    o_ref[...] = base + m0 + m1 + bias


def kernel(x, edge_index, W_lin, b_lin, W_s2t, b_s2t, W_t2s, b_t2s):
    n, d = x.shape
    e = edge_index.shape[1]
    chunk = 80
    assert e % (N_TILES * chunk) == 0 and d == 128
    n_chunks = e // (N_TILES * chunk)
    assert n_chunks % 2 == 0

    eidx = edge_index.astype(jnp.int32).reshape(2, N_TILES, n_chunks, chunk)
    n_pad = ((n + 16 * N_TILES - 1) // (16 * N_TILES)) * (16 * N_TILES)
    zrows = jnp.zeros((n_pad // N_TILES, d), jnp.bfloat16)

    agg, cnts = _sc_aggregate(x.astype(jnp.bfloat16), eidx, zrows, n_pad=n_pad, chunk=chunk,
                              n_chunks=n_chunks)
    cnts = cnts.reshape(2, N_TILES, n_pad).transpose(0, 2, 1)

    blk = 2000
    grid = (n // blk,)
    out = pl.pallas_call(
        _tc_combine_body,
        grid=grid,
        in_specs=[
            pl.BlockSpec((blk, d), lambda i: (i, 0)),
            pl.BlockSpec((blk, d), lambda i: (i, 0)),
            pl.BlockSpec((blk, d), lambda i: (i, 0)),
            pl.BlockSpec((blk, N_TILES), lambda i: (i, 0)),
            pl.BlockSpec((blk, N_TILES), lambda i: (i, 0)),
            pl.BlockSpec((d, d), lambda i: (0, 0)),
            pl.BlockSpec((d, d), lambda i: (0, 0)),
            pl.BlockSpec((d, d), lambda i: (0, 0)),
            pl.BlockSpec((1, d), lambda i: (0, 0)),
            pl.BlockSpec((1, d), lambda i: (0, 0)),
            pl.BlockSpec((1, d), lambda i: (0, 0)),
        ],
        out_specs=pl.BlockSpec((blk, d), lambda i: (i, 0)),
        out_shape=jax.ShapeDtypeStruct((n, d), jnp.float32),
    )(x, agg[0], agg[1], cnts[0], cnts[1], W_lin, W_s2t, W_t2s,
      b_lin.reshape(1, d), b_s2t.reshape(1, d), b_t2s.reshape(1, d))
    return out
